# Initial kernel scaffold; baseline (speedup 1.0000x reference)
import functools

import jax
import jax.numpy as jnp
from jax import lax
from jax.experimental import pallas as pl
from jax.experimental.pallas import tpu as pltpu
from jax.experimental.pallas import tpu_sc as plsc

_NC = 2
_NS = 16
_CHUNK = 128


@functools.partial(jax.jit, static_argnums=(3, 4))
def _pos_encode(xf, pf, W, N, D):
    NW = _NC * _NS
    n_per_w = N // NW
    n_chunks = n_per_w // _CHUNK
    mesh = plsc.VectorSubcoreMesh(
        core_axis_name="c", subcore_axis_name="s",
        num_cores=_NC, num_subcores=_NS,
    )

    @functools.partial(
        pl.kernel,
        out_type=jax.ShapeDtypeStruct((N, D), jnp.float32),
        mesh=mesh,
        scratch_types=[
            pltpu.VMEM((n_per_w,), jnp.int32),
            pltpu.VMEM((_CHUNK, D), jnp.float32),
            pltpu.SemaphoreType.DMA,
        ],
    )
    def k(x_hbm, idx_hbm, w_hbm, out_hbm, idx_v, buf, sem):
        wid = lax.axis_index("s") * _NC + lax.axis_index("c")
        base = wid * n_per_w
        pltpu.sync_copy(idx_hbm.at[pl.ds(base, n_per_w)], idx_v)

        def body(ch, carry):
            o = base + ch * _CHUNK
            pltpu.sync_copy(x_hbm.at[pl.ds(o, _CHUNK)], buf)
            pltpu.async_copy(
                w_hbm.at[idx_v.at[pl.ds(ch * _CHUNK, _CHUNK)]], buf, sem, add=True
            ).wait()
            pltpu.sync_copy(buf, out_hbm.at[pl.ds(o, _CHUNK)])
            return carry

        lax.fori_loop(0, n_chunks, body, 0)

    return k(xf, pf, W)


def kernel(x, positions, W):
    B, L, D = x.shape
    N = B * L
    xf = x.reshape(N, D)
    pf = positions.reshape(N).astype(jnp.int32)
    out = _pos_encode(xf, pf, W, N, D)
    return out.reshape(B, L, D)


# speedup vs baseline: 1.3635x; 1.3635x over previous
import functools

import jax
import jax.numpy as jnp
from jax import lax
from jax.experimental import pallas as pl
from jax.experimental.pallas import tpu as pltpu
from jax.experimental.pallas import tpu_sc as plsc

_NC = 2
_NS = 16
_CHUNK = 64
_LANES = 16


@functools.partial(jax.jit, static_argnums=(3, 4))
def _pos_encode(xf, pf, W, N, D):
    NW = _NC * _NS
    n_per_w = N // NW
    n_chunks = n_per_w // _CHUNK
    n_col = D // _LANES
    mesh = plsc.VectorSubcoreMesh(
        core_axis_name="c", subcore_axis_name="s",
        num_cores=_NC, num_subcores=_NS,
    )

    @functools.partial(
        pl.kernel,
        out_type=jax.ShapeDtypeStruct((N, D), jnp.float32),
        mesh=mesh,
        scratch_types=[
            pltpu.VMEM((n_per_w,), jnp.int32),
            pltpu.VMEM((_CHUNK, D), jnp.float32),
            pltpu.VMEM((_CHUNK, D), jnp.float32),
            pltpu.SemaphoreType.DMA,
        ],
    )
    def k(x_hbm, idx_hbm, w_hbm, out_hbm, idx_v, xbuf, wbuf, sem):
        wid = lax.axis_index("s") * _NC + lax.axis_index("c")
        base = wid * n_per_w
        pltpu.sync_copy(idx_hbm.at[pl.ds(base, n_per_w)], idx_v)

        def body(ch, carry):
            o = base + ch * _CHUNK
            gather = pltpu.async_copy(
                w_hbm.at[idx_v.at[pl.ds(ch * _CHUNK, _CHUNK)]], wbuf, sem
            )
            pltpu.sync_copy(x_hbm.at[pl.ds(o, _CHUNK)], xbuf)
            gather.wait()

            @plsc.parallel_loop(0, _CHUNK, unroll=2)
            def _(r):
                for c in range(n_col):
                    plsc.addupdate(
                        xbuf.at[r, pl.ds(c * _LANES, _LANES)],
                        wbuf[r, pl.ds(c * _LANES, _LANES)],
                    )

            pltpu.sync_copy(xbuf, out_hbm.at[pl.ds(o, _CHUNK)])
            return carry

        lax.fori_loop(0, n_chunks, body, 0)

    return k(xf, pf, W)


def kernel(x, positions, W):
    B, L, D = x.shape
    N = B * L
    xf = x.reshape(N, D)
    pf = positions.reshape(N).astype(jnp.int32)
    out = _pos_encode(xf, pf, W, N, D)
    return out.reshape(B, L, D)
